# Initial kernel scaffold; baseline (speedup 1.0000x reference)
import jax
import jax.numpy as jnp
from jax.experimental import pallas as pl


def kernel(x, weights, bias):
    raise NotImplementedError("write your pallas kernel here")


# speedup vs baseline: 1.8000x; 1.8000x over previous
import functools

import jax
import jax.numpy as jnp
from jax import lax
from jax.experimental import pallas as pl
from jax.experimental.pallas import tpu as pltpu

ROWS = 8192
DIM = 2048
NUM_EXPERTS = 64
K = 8
BM = 1024


def _gate_kernel(x_ref, wt_ref, bias_ref, w_ref, idx_ref):
    logits = jnp.dot(
        x_ref[...], wt_ref[...],
        preferred_element_type=jnp.float32,
    )
    m = jnp.max(logits, axis=1, keepdims=True)
    e = jnp.exp(logits - m)
    p = e / jnp.sum(e, axis=1, keepdims=True)
    score = p + bias_ref[...]

    iota = lax.broadcasted_iota(jnp.int32, (BM, NUM_EXPERTS), 1)
    work = score
    neg_inf = jnp.float32(-jnp.inf)
    for k in range(K):
        mk = jnp.max(work, axis=1, keepdims=True)
        is_max = work == mk
        ik = jnp.min(jnp.where(is_max, iota, NUM_EXPERTS), axis=1, keepdims=True)
        w_ref[:, k : k + 1] = mk
        idx_ref[:, k : k + 1] = ik
        work = jnp.where(iota == ik, neg_inf, work)


@jax.jit
def kernel(x, weights, bias):
    wt = weights.T
    bias2 = bias.reshape(1, NUM_EXPERTS)
    grid = (ROWS // BM,)
    w, idx = pl.pallas_call(
        _gate_kernel,
        grid=grid,
        in_specs=[
            pl.BlockSpec((BM, DIM), lambda i: (i, 0)),
            pl.BlockSpec((DIM, NUM_EXPERTS), lambda i: (0, 0)),
            pl.BlockSpec((1, NUM_EXPERTS), lambda i: (0, 0)),
        ],
        out_specs=[
            pl.BlockSpec((BM, K), lambda i: (i, 0)),
            pl.BlockSpec((BM, K), lambda i: (i, 0)),
        ],
        out_shape=[
            jax.ShapeDtypeStruct((ROWS, K), jnp.float32),
            jax.ShapeDtypeStruct((ROWS, K), jnp.int32),
        ],
    )(x, wt, bias2)
    return w, idx
